# Initial kernel scaffold; baseline (speedup 1.0000x reference)
import jax
import jax.numpy as jnp
from jax.experimental import pallas as pl


def kernel(node_indices, edge_index, edge_type, node_features, basis1, comp1, root1, bias1, basis2, comp2, root2, bias2, Wc1, bc1, Wc2, bc2):
    raise NotImplementedError("write your pallas kernel here")


# speedup vs baseline: 4.8395x; 4.8395x over previous
import functools

import jax
import jax.numpy as jnp
from jax import lax
from jax.experimental import pallas as pl
from jax.experimental.pallas import tpu as pltpu
from jax.experimental.pallas import tpu_sc as plsc

N_NODES = 10000
D_IN = 128
D_HID = 128
D_EXP = 64
N_REL = 2
N_EDGES = 320000
N_BATCH = 4096

NC = 2
NS = 16

NPAD = 10112
ROWS_PER_TILE = NPAD // NS
CHUNK = 128
N_CHUNKS = N_EDGES // CHUNK
BASE_CHUNKS = N_CHUNKS // NS
EXTRA_TILES = N_CHUNKS - BASE_CHUNKS * NS


def _sc_mesh():
    return plsc.VectorSubcoreMesh(
        core_axis_name="c", subcore_axis_name="s", num_cores=NC, num_subcores=NS
    )


def _edge_kernel_body(with_counts, d_model, src_hbm, dst_hbm, typ_hbm, y_hbm,
                      z_big, z_small, ones_hbm, *refs):
    if with_counts:
        (s_out, c_out, acc_sh, cnt_sh, src_v, dst_v, typ_v, g_v, d_v, rows_v,
         ones_v, sem) = refs
    else:
        (s_out, acc_sh, src_v, dst_v, typ_v, g_v, d_v, rows_v, sem) = refs
        c_out = cnt_sh = ones_v = None

    c = lax.axis_index("c")
    s = lax.axis_index("s")

    r0 = s * ROWS_PER_TILE
    pltpu.sync_copy(z_big.at[pl.ds(r0, ROWS_PER_TILE)],
                    acc_sh.at[pl.ds(r0, ROWS_PER_TILE)])
    if with_counts:
        pltpu.sync_copy(z_small.at[pl.ds(r0, ROWS_PER_TILE)],
                        cnt_sh.at[pl.ds(r0, ROWS_PER_TILE)])
        pltpu.sync_copy(ones_hbm, ones_v)
    plsc.subcore_barrier()

    lane = lax.iota(jnp.int32, 16)
    n_chunks_here = BASE_CHUNKS + (s < EXTRA_TILES).astype(jnp.int32)

    def chunk_body(j, _):
        base = (s + NS * j) * CHUNK
        pltpu.sync_copy(src_hbm.at[pl.ds(base, CHUNK)], src_v)
        pltpu.sync_copy(dst_hbm.at[pl.ds(base, CHUNK)], dst_v)
        pltpu.sync_copy(typ_hbm.at[pl.ds(base, CHUNK)], typ_v)

        def vec_body(jj, _):
            sl = pl.ds(jj * 16, 16)
            sv = src_v[sl]
            dv = dst_v[sl]
            tv = typ_v[sl]
            g_v[sl] = sv + c * N_NODES
            trash = N_NODES + ((s * 13 + jj * 16) % 96) + lane
            d_v[sl] = jnp.where(tv == c, dv, trash)
            return 0

        lax.fori_loop(0, CHUNK // 16, vec_body, 0)
        pltpu.async_copy(y_hbm.at[g_v], rows_v, sem).wait()
        pltpu.sync_copy(rows_v, acc_sh.at[d_v], add=True)
        if with_counts:
            pltpu.sync_copy(ones_v, cnt_sh.at[d_v], add=True)
        return 0

    lax.fori_loop(0, n_chunks_here, chunk_body, 0)
    plsc.subcore_barrier()

    out0 = c * NPAD + r0
    pltpu.sync_copy(acc_sh.at[pl.ds(r0, ROWS_PER_TILE)],
                    s_out.at[pl.ds(out0, ROWS_PER_TILE)])
    if with_counts:
        pltpu.sync_copy(cnt_sh.at[pl.ds(r0, ROWS_PER_TILE)],
                        c_out.at[pl.ds(out0, ROWS_PER_TILE)])


def _make_edge_kernel(with_counts, d_model):
    out_type = [jax.ShapeDtypeStruct((NC * NPAD, d_model), jnp.float32)]
    scratch = [
        pltpu.VMEM_SHARED((NPAD, d_model), jnp.float32),
    ]
    if with_counts:
        out_type.append(jax.ShapeDtypeStruct((NC * NPAD, 16), jnp.float32))
        scratch.append(pltpu.VMEM_SHARED((NPAD, 16), jnp.float32))
    scratch += [
        pltpu.VMEM((CHUNK,), jnp.int32),
        pltpu.VMEM((CHUNK,), jnp.int32),
        pltpu.VMEM((CHUNK,), jnp.int32),
        pltpu.VMEM((CHUNK,), jnp.int32),
        pltpu.VMEM((CHUNK,), jnp.int32),
        pltpu.VMEM((CHUNK, d_model), jnp.float32),
    ]
    if with_counts:
        scratch.append(pltpu.VMEM((CHUNK, 16), jnp.float32))
    scratch.append(pltpu.SemaphoreType.DMA)

    if with_counts:
        out_type = tuple(out_type)
    else:
        out_type = out_type[0]

    def ordered_body(*args):
        if with_counts:
            (src_hbm, dst_hbm, typ_hbm, y_hbm, z_big, z_small, ones_hbm,
             s_out, c_out, acc_sh, cnt_sh, src_v, dst_v, typ_v, g_v, d_v,
             rows_v, ones_v, sem) = args
            _edge_kernel_body(True, d_model, src_hbm, dst_hbm, typ_hbm, y_hbm,
                              z_big, z_small, ones_hbm, s_out, c_out, acc_sh,
                              cnt_sh, src_v, dst_v, typ_v, g_v, d_v, rows_v,
                              ones_v, sem)
        else:
            (src_hbm, dst_hbm, typ_hbm, y_hbm, z_big,
             s_out, acc_sh, src_v, dst_v, typ_v, g_v, d_v, rows_v, sem) = args
            _edge_kernel_body(False, d_model, src_hbm, dst_hbm, typ_hbm, y_hbm,
                              z_big, None, None, s_out, acc_sh, src_v, dst_v,
                              typ_v, g_v, d_v, rows_v, sem)

    return pl.kernel(ordered_body, out_type=out_type, mesh=_sc_mesh(),
                     scratch_types=scratch,
                     compiler_params=pltpu.CompilerParams(
                         use_tc_tiling_on_sc=False))


def _gather_kernel_body(table_hbm, idx_hbm, out_hbm, idx_v, rows_v, sem):
    wid = lax.axis_index("s") * NC + lax.axis_index("c")
    per_w = N_BATCH // (NC * NS)
    base = wid * per_w
    pltpu.sync_copy(idx_hbm.at[pl.ds(base, per_w)], idx_v)
    pltpu.async_copy(table_hbm.at[idx_v], rows_v, sem).wait()
    pltpu.sync_copy(rows_v, out_hbm.at[pl.ds(base, per_w)])


def _make_gather_kernel():
    per_w = N_BATCH // (NC * NS)
    return pl.kernel(
        _gather_kernel_body,
        out_type=jax.ShapeDtypeStruct((N_BATCH, D_EXP), jnp.float32),
        mesh=_sc_mesh(),
        scratch_types=[
            pltpu.VMEM((per_w,), jnp.int32),
            pltpu.VMEM((per_w, D_EXP), jnp.float32),
            pltpu.SemaphoreType.DMA,
        ],
        compiler_params=pltpu.CompilerParams(use_tc_tiling_on_sc=False),
    )


_BN = 1000


def _transform1_body(x_ref, basis_ref, comp_ref, root_ref, bias_ref,
                     y_ref, dense_ref):
    x = x_ref[...]
    w0 = comp_ref[0, 0] * basis_ref[0] + comp_ref[0, 1] * basis_ref[1]
    w1 = comp_ref[1, 0] * basis_ref[0] + comp_ref[1, 1] * basis_ref[1]
    y_ref[0] = jnp.dot(x, w0, preferred_element_type=jnp.float32)
    y_ref[1] = jnp.dot(x, w1, preferred_element_type=jnp.float32)
    dense_ref[...] = (
        jnp.dot(x, root_ref[...], preferred_element_type=jnp.float32)
        + bias_ref[...]
    )


def _transform1(x, basis1, comp1, root1, bias1):
    grid = N_NODES // _BN
    return pl.pallas_call(
        _transform1_body,
        grid=(grid,),
        in_specs=[
            pl.BlockSpec((_BN, D_IN), lambda i: (i, 0)),
            pl.BlockSpec((N_REL, D_IN, D_HID), lambda i: (0, 0, 0)),
            pl.BlockSpec(memory_space=pltpu.SMEM),
            pl.BlockSpec((D_IN, D_HID), lambda i: (0, 0)),
            pl.BlockSpec((1, D_HID), lambda i: (0, 0)),
        ],
        out_specs=[
            pl.BlockSpec((N_REL, _BN, D_HID), lambda i: (0, i, 0)),
            pl.BlockSpec((_BN, D_HID), lambda i: (i, 0)),
        ],
        out_shape=[
            jax.ShapeDtypeStruct((N_REL, N_NODES, D_HID), jnp.float32),
            jax.ShapeDtypeStruct((N_NODES, D_HID), jnp.float32),
        ],
    )(x, basis1, comp1, root1, bias1)


def _combine2_body(dense_ref, s0_ref, s1_ref, c0_ref, c1_ref,
                   basis_ref, comp_ref, root_ref, bias_ref,
                   y_ref, dense2_ref):
    h = (dense_ref[...]
         + s0_ref[...] / jnp.maximum(c0_ref[...], 1.0)
         + s1_ref[...] / jnp.maximum(c1_ref[...], 1.0))
    h = jnp.maximum(h, 0.0)
    w0 = comp_ref[0, 0] * basis_ref[0] + comp_ref[0, 1] * basis_ref[1]
    w1 = comp_ref[1, 0] * basis_ref[0] + comp_ref[1, 1] * basis_ref[1]
    y_ref[0] = jnp.dot(h, w0, preferred_element_type=jnp.float32)
    y_ref[1] = jnp.dot(h, w1, preferred_element_type=jnp.float32)
    dense2_ref[...] = (
        jnp.dot(h, root_ref[...], preferred_element_type=jnp.float32)
        + bias_ref[...]
    )


def _combine2(dense1, s0, s1, c0, c1, basis2, comp2, root2, bias2):
    grid = N_NODES // _BN
    return pl.pallas_call(
        _combine2_body,
        grid=(grid,),
        in_specs=[
            pl.BlockSpec((_BN, D_HID), lambda i: (i, 0)),
            pl.BlockSpec((_BN, D_HID), lambda i: (i, 0)),
            pl.BlockSpec((_BN, D_HID), lambda i: (i, 0)),
            pl.BlockSpec((_BN, 1), lambda i: (i, 0)),
            pl.BlockSpec((_BN, 1), lambda i: (i, 0)),
            pl.BlockSpec((N_REL, D_HID, D_EXP), lambda i: (0, 0, 0)),
            pl.BlockSpec(memory_space=pltpu.SMEM),
            pl.BlockSpec((D_HID, D_EXP), lambda i: (0, 0)),
            pl.BlockSpec((1, D_EXP), lambda i: (0, 0)),
        ],
        out_specs=[
            pl.BlockSpec((N_REL, _BN, D_EXP), lambda i: (0, i, 0)),
            pl.BlockSpec((_BN, D_EXP), lambda i: (i, 0)),
        ],
        out_shape=[
            jax.ShapeDtypeStruct((N_REL, N_NODES, D_EXP), jnp.float32),
            jax.ShapeDtypeStruct((N_NODES, D_EXP), jnp.float32),
        ],
    )(dense1, s0, s1, c0, c1, basis2, comp2, root2, bias2)


def _final_combine_body(dense_ref, s0_ref, s1_ref, c0_ref, c1_ref, out_ref):
    out_ref[...] = (dense_ref[...]
                    + s0_ref[...] / jnp.maximum(c0_ref[...], 1.0)
                    + s1_ref[...] / jnp.maximum(c1_ref[...], 1.0))


def _final_combine(dense2, s0, s1, c0, c1):
    grid = N_NODES // _BN
    return pl.pallas_call(
        _final_combine_body,
        grid=(grid,),
        in_specs=[
            pl.BlockSpec((_BN, D_EXP), lambda i: (i, 0)),
            pl.BlockSpec((_BN, D_EXP), lambda i: (i, 0)),
            pl.BlockSpec((_BN, D_EXP), lambda i: (i, 0)),
            pl.BlockSpec((_BN, 1), lambda i: (i, 0)),
            pl.BlockSpec((_BN, 1), lambda i: (i, 0)),
        ],
        out_specs=pl.BlockSpec((_BN, D_EXP), lambda i: (i, 0)),
        out_shape=jax.ShapeDtypeStruct((N_NODES, D_EXP), jnp.float32),
    )(dense2, s0, s1, c0, c1)


def _classifier_body(x_ref, w1_ref, b1_ref, w2_ref, b2_ref, out_ref):
    h = jnp.dot(x_ref[...], w1_ref[...], preferred_element_type=jnp.float32)
    h = jnp.maximum(h + b1_ref[...], 0.0)
    z = jnp.dot(h, w2_ref[...], preferred_element_type=jnp.float32)
    out_ref[...] = jax.nn.sigmoid(z + b2_ref[...])


def _classifier(batch_repr, Wc1, bc1, Wc2, bc2):
    return pl.pallas_call(
        _classifier_body,
        in_specs=[
            pl.BlockSpec((N_BATCH, D_EXP), lambda: (0, 0)),
            pl.BlockSpec((D_EXP, 32), lambda: (0, 0)),
            pl.BlockSpec((1, 32), lambda: (0, 0)),
            pl.BlockSpec((32, 1), lambda: (0, 0)),
            pl.BlockSpec((1, 1), lambda: (0, 0)),
        ],
        out_specs=pl.BlockSpec((N_BATCH, 1), lambda: (0, 0)),
        out_shape=jax.ShapeDtypeStruct((N_BATCH, 1), jnp.float32),
    )(batch_repr, Wc1, bc1, Wc2, bc2)


def kernel(node_indices, edge_index, edge_type, node_features, basis1, comp1,
           root1, bias1, basis2, comp2, root2, bias2, Wc1, bc1, Wc2, bc2):
    src = edge_index[0]
    dst = edge_index[1]
    typ = edge_type

    z128 = jnp.zeros((NPAD, D_HID), jnp.float32)
    z64 = jnp.zeros((NPAD, D_EXP), jnp.float32)
    z16 = jnp.zeros((NPAD, 16), jnp.float32)
    ones = jnp.ones((CHUNK, 16), jnp.float32)

    y1, dense1 = _transform1(node_features, basis1, comp1, root1,
                             bias1.reshape(1, D_HID))
    y1_flat = y1.reshape(N_REL * N_NODES, D_HID)
    s1_flat, cnt_flat = _make_edge_kernel(True, D_HID)(
        src, dst, typ, y1_flat, z128, z16, ones)
    s10 = s1_flat[:N_NODES]
    s11 = s1_flat[NPAD:NPAD + N_NODES]
    c0 = cnt_flat[:N_NODES, :1]
    c1 = cnt_flat[NPAD:NPAD + N_NODES, :1]

    y2, dense2 = _combine2(dense1, s10, s11, c0, c1, basis2, comp2, root2,
                           bias2.reshape(1, D_EXP))
    y2_flat = y2.reshape(N_REL * N_NODES, D_EXP)
    s2_flat = _make_edge_kernel(False, D_EXP)(src, dst, typ, y2_flat, z64)
    s20 = s2_flat[:N_NODES]
    s21 = s2_flat[NPAD:NPAD + N_NODES]

    out2 = _final_combine(dense2, s20, s21, c0, c1)

    batch_repr = _make_gather_kernel()(out2, node_indices)
    bot_prob = _classifier(batch_repr, Wc1, bc1.reshape(1, 32), Wc2,
                           bc2.reshape(1, 1))
    return (batch_repr, bot_prob)


# speedup vs baseline: 11.3469x; 2.3446x over previous
import functools

import jax
import jax.numpy as jnp
from jax import lax
from jax.experimental import pallas as pl
from jax.experimental.pallas import tpu as pltpu
from jax.experimental.pallas import tpu_sc as plsc

N_NODES = 10000
D_IN = 128
D_HID = 128
D_EXP = 64
N_REL = 2
N_EDGES = 320000
N_BATCH = 4096

NC = 2
NS = 16

NPAD = 10112
ROWS_PER_TILE = NPAD // NS
CHUNK = 128
EPT = N_EDGES // NS
NCH = EPT // CHUNK
TAIL = EPT - NCH * CHUNK
NBUF1 = 2
NBUF2 = 6


def _sc_mesh():
    return plsc.VectorSubcoreMesh(
        core_axis_name="c", subcore_axis_name="s", num_cores=NC, num_subcores=NS
    )


def _make_edge_kernel(with_counts, d_model, nbuf):
    out_type = [jax.ShapeDtypeStruct((NC * NPAD, d_model), jnp.float32)]
    scratch = [pltpu.VMEM_SHARED((NPAD, d_model), jnp.float32)]
    if with_counts:
        out_type.append(jax.ShapeDtypeStruct((NC * NPAD, 16), jnp.float32))
        scratch.append(pltpu.VMEM_SHARED((NPAD, 16), jnp.float32))
    scratch += [pltpu.VMEM((3, CHUNK), jnp.int32) for _ in range(nbuf)]
    scratch += [pltpu.VMEM((CHUNK,), jnp.int32) for _ in range(nbuf)]
    scratch += [pltpu.VMEM((CHUNK,), jnp.int32) for _ in range(nbuf)]
    scratch += [pltpu.VMEM((CHUNK, d_model), jnp.float32)
                for _ in range(nbuf)]
    if with_counts:
        scratch.append(pltpu.VMEM((CHUNK, 16), jnp.float32))
    scratch += [pltpu.VMEM((3, TAIL), jnp.int32),
                pltpu.VMEM((TAIL,), jnp.int32),
                pltpu.VMEM((TAIL,), jnp.int32)]
    n_sem_kinds = 4 if with_counts else 3
    scratch += [pltpu.SemaphoreType.DMA] * (nbuf * n_sem_kinds + 1)

    out_type = tuple(out_type) if with_counts else out_type[0]

    def body(*args):
        if with_counts:
            e3_hbm, y_hbm, z_big, z_small, ones_hbm, s_out, c_out = args[:7]
            rest = list(args[7:])
        else:
            e3_hbm, y_hbm, z_big, s_out = args[:4]
            c_out = z_small = ones_hbm = None
            rest = list(args[4:])
        it = iter(rest)
        acc_sh = next(it)
        cnt_sh = next(it) if with_counts else None
        e3_v = [next(it) for _ in range(nbuf)]
        g_v = [next(it) for _ in range(nbuf)]
        d_v = [next(it) for _ in range(nbuf)]
        rows_v = [next(it) for _ in range(nbuf)]
        ones_v = next(it) if with_counts else None
        e3_t = next(it)
        g_t = next(it)
        d_t = next(it)
        sem_i = [next(it) for _ in range(nbuf)]
        sem_g = [next(it) for _ in range(nbuf)]
        sem_s = [next(it) for _ in range(nbuf)]
        sem_c = [next(it) for _ in range(nbuf)] if with_counts else None
        sem_t = next(it)

        c = lax.axis_index("c")
        s = lax.axis_index("s")
        ebase = s * EPT
        lane = lax.iota(jnp.int32, 16)

        r0 = s * ROWS_PER_TILE
        pltpu.sync_copy(z_big.at[pl.ds(r0, ROWS_PER_TILE)],
                        acc_sh.at[pl.ds(r0, ROWS_PER_TILE)])
        if with_counts:
            pltpu.sync_copy(z_small.at[pl.ds(r0, ROWS_PER_TILE)],
                            cnt_sh.at[pl.ds(r0, ROWS_PER_TILE)])
            pltpu.sync_copy(ones_hbm, ones_v)
        plsc.subcore_barrier()

        def idx_src(cj):
            base = ebase + jnp.minimum(cj, NCH - 1) * CHUNK
            return e3_hbm.at[:, pl.ds(base, CHUNK)]

        def issue_idx(cj, b):
            pltpu.async_copy(idx_src(cj), e3_v[b], sem_i[b])

        def wait_idx(b):
            pltpu.make_async_copy(idx_src(0), e3_v[b], sem_i[b]).wait()

        def build(b, e3r, gr, dr, ngrp):
            for jj in range(ngrp):
                sl = pl.ds(jj * 16, 16)
                gr[sl] = e3r[0, sl] + c * N_NODES
                trash = N_NODES + ((s * 13 + jj * 16) % 96) + lane
                dr[sl] = jnp.where(e3r[2, sl] == c, e3r[1, sl], trash)

        class _Op:
            def __init__(self, src, dst, sem, add=False):
                self.a = (src, dst, sem)
                self.add = add

            def start(self):
                pltpu.async_copy(*self.a, add=self.add)

            def wait(self):
                pltpu.make_async_copy(*self.a).wait()

        def gather_desc(b):
            return _Op(y_hbm.at[g_v[b]], rows_v[b], sem_g[b])

        def scat_desc(b):
            return _Op(rows_v[b], acc_sh.at[d_v[b]], sem_s[b], add=True)

        def cnt_desc(b):
            return _Op(ones_v, cnt_sh.at[d_v[b]], sem_c[b], add=True)

        for b in range(nbuf):
            issue_idx(b, b)
        for b in range(nbuf):
            wait_idx(b)
            build(b, e3_v[b], g_v[b], d_v[b], CHUNK // 16)
            gather_desc(b).start()
            issue_idx(nbuf + b, b)

        def grp(j, _):
            for b in range(nbuf):
                gather_desc(b).wait()
                scat_desc(b).start()
                if with_counts:
                    cnt_desc(b).start()
            for b in range(nbuf):
                wait_idx(b)
                scat_desc(b).wait()
                if with_counts:
                    cnt_desc(b).wait()
                build(b, e3_v[b], g_v[b], d_v[b], CHUNK // 16)
                gather_desc(b).start()
                issue_idx((j + 2) * nbuf + b, b)
            return 0

        lax.fori_loop(0, NCH // nbuf - 1, grp, 0)

        for b in range(nbuf):
            gather_desc(b).wait()
            scat_desc(b).start()
            if with_counts:
                cnt_desc(b).start()
        for b in range(nbuf):
            wait_idx(b)
            scat_desc(b).wait()
            if with_counts:
                cnt_desc(b).wait()

        rows_t = rows_v[0].at[pl.ds(0, TAIL)]
        pltpu.sync_copy(e3_hbm.at[:, pl.ds(ebase + NCH * CHUNK, TAIL)], e3_t)
        build(None, e3_t, g_t, d_t, TAIL // 16)
        pltpu.async_copy(y_hbm.at[g_t], rows_t, sem_t).wait()
        pltpu.sync_copy(rows_t, acc_sh.at[d_t], add=True)
        if with_counts:
            pltpu.sync_copy(ones_v.at[pl.ds(0, TAIL)], cnt_sh.at[d_t],
                            add=True)

        plsc.subcore_barrier()

        out0 = c * NPAD + r0
        pltpu.sync_copy(acc_sh.at[pl.ds(r0, ROWS_PER_TILE)],
                        s_out.at[pl.ds(out0, ROWS_PER_TILE)])
        if with_counts:
            pltpu.sync_copy(cnt_sh.at[pl.ds(r0, ROWS_PER_TILE)],
                            c_out.at[pl.ds(out0, ROWS_PER_TILE)])

    return pl.kernel(body, out_type=out_type, mesh=_sc_mesh(),
                     scratch_types=scratch,
                     compiler_params=pltpu.CompilerParams(
                         use_tc_tiling_on_sc=False))


def _gather_kernel_body(table_hbm, idx_hbm, out_hbm, idx_v, rows_v, sem):
    wid = lax.axis_index("s") * NC + lax.axis_index("c")
    per_w = N_BATCH // (NC * NS)
    base = wid * per_w
    pltpu.sync_copy(idx_hbm.at[pl.ds(base, per_w)], idx_v)
    pltpu.async_copy(table_hbm.at[idx_v], rows_v, sem).wait()
    pltpu.sync_copy(rows_v, out_hbm.at[pl.ds(base, per_w)])


def _make_gather_kernel():
    per_w = N_BATCH // (NC * NS)
    return pl.kernel(
        _gather_kernel_body,
        out_type=jax.ShapeDtypeStruct((N_BATCH, D_EXP), jnp.float32),
        mesh=_sc_mesh(),
        scratch_types=[
            pltpu.VMEM((per_w,), jnp.int32),
            pltpu.VMEM((per_w, D_EXP), jnp.float32),
            pltpu.SemaphoreType.DMA,
        ],
        compiler_params=pltpu.CompilerParams(use_tc_tiling_on_sc=False),
    )


_BN = 1000


def _transform1_body(x_ref, basis_ref, comp_ref, root_ref, bias_ref,
                     y_ref, dense_ref):
    x = x_ref[...]
    w0 = comp_ref[0, 0] * basis_ref[0] + comp_ref[0, 1] * basis_ref[1]
    w1 = comp_ref[1, 0] * basis_ref[0] + comp_ref[1, 1] * basis_ref[1]
    y_ref[0] = jnp.dot(x, w0, preferred_element_type=jnp.float32)
    y_ref[1] = jnp.dot(x, w1, preferred_element_type=jnp.float32)
    dense_ref[...] = (
        jnp.dot(x, root_ref[...], preferred_element_type=jnp.float32)
        + bias_ref[...]
    )


def _transform1(x, basis1, comp1, root1, bias1):
    grid = N_NODES // _BN
    return pl.pallas_call(
        _transform1_body,
        grid=(grid,),
        in_specs=[
            pl.BlockSpec((_BN, D_IN), lambda i: (i, 0)),
            pl.BlockSpec((N_REL, D_IN, D_HID), lambda i: (0, 0, 0)),
            pl.BlockSpec(memory_space=pltpu.SMEM),
            pl.BlockSpec((D_IN, D_HID), lambda i: (0, 0)),
            pl.BlockSpec((1, D_HID), lambda i: (0, 0)),
        ],
        out_specs=[
            pl.BlockSpec((N_REL, _BN, D_HID), lambda i: (0, i, 0)),
            pl.BlockSpec((_BN, D_HID), lambda i: (i, 0)),
        ],
        out_shape=[
            jax.ShapeDtypeStruct((N_REL, N_NODES, D_HID), jnp.float32),
            jax.ShapeDtypeStruct((N_NODES, D_HID), jnp.float32),
        ],
    )(x, basis1, comp1, root1, bias1)


def _combine2_body(dense_ref, s0_ref, s1_ref, c0_ref, c1_ref,
                   basis_ref, comp_ref, root_ref, bias_ref,
                   y_ref, dense2_ref):
    h = (dense_ref[...]
         + s0_ref[...] / jnp.maximum(c0_ref[...], 1.0)
         + s1_ref[...] / jnp.maximum(c1_ref[...], 1.0))
    h = jnp.maximum(h, 0.0)
    w0 = comp_ref[0, 0] * basis_ref[0] + comp_ref[0, 1] * basis_ref[1]
    w1 = comp_ref[1, 0] * basis_ref[0] + comp_ref[1, 1] * basis_ref[1]
    y_ref[0] = jnp.dot(h, w0, preferred_element_type=jnp.float32)
    y_ref[1] = jnp.dot(h, w1, preferred_element_type=jnp.float32)
    dense2_ref[...] = (
        jnp.dot(h, root_ref[...], preferred_element_type=jnp.float32)
        + bias_ref[...]
    )


def _combine2(dense1, s0, s1, c0, c1, basis2, comp2, root2, bias2):
    grid = N_NODES // _BN
    return pl.pallas_call(
        _combine2_body,
        grid=(grid,),
        in_specs=[
            pl.BlockSpec((_BN, D_HID), lambda i: (i, 0)),
            pl.BlockSpec((_BN, D_HID), lambda i: (i, 0)),
            pl.BlockSpec((_BN, D_HID), lambda i: (i, 0)),
            pl.BlockSpec((_BN, 1), lambda i: (i, 0)),
            pl.BlockSpec((_BN, 1), lambda i: (i, 0)),
            pl.BlockSpec((N_REL, D_HID, D_EXP), lambda i: (0, 0, 0)),
            pl.BlockSpec(memory_space=pltpu.SMEM),
            pl.BlockSpec((D_HID, D_EXP), lambda i: (0, 0)),
            pl.BlockSpec((1, D_EXP), lambda i: (0, 0)),
        ],
        out_specs=[
            pl.BlockSpec((N_REL, _BN, D_EXP), lambda i: (0, i, 0)),
            pl.BlockSpec((_BN, D_EXP), lambda i: (i, 0)),
        ],
        out_shape=[
            jax.ShapeDtypeStruct((N_REL, N_NODES, D_EXP), jnp.float32),
            jax.ShapeDtypeStruct((N_NODES, D_EXP), jnp.float32),
        ],
    )(dense1, s0, s1, c0, c1, basis2, comp2, root2, bias2)


def _final_combine_body(dense_ref, s0_ref, s1_ref, c0_ref, c1_ref, out_ref):
    out_ref[...] = (dense_ref[...]
                    + s0_ref[...] / jnp.maximum(c0_ref[...], 1.0)
                    + s1_ref[...] / jnp.maximum(c1_ref[...], 1.0))


def _final_combine(dense2, s0, s1, c0, c1):
    grid = N_NODES // _BN
    return pl.pallas_call(
        _final_combine_body,
        grid=(grid,),
        in_specs=[
            pl.BlockSpec((_BN, D_EXP), lambda i: (i, 0)),
            pl.BlockSpec((_BN, D_EXP), lambda i: (i, 0)),
            pl.BlockSpec((_BN, D_EXP), lambda i: (i, 0)),
            pl.BlockSpec((_BN, 1), lambda i: (i, 0)),
            pl.BlockSpec((_BN, 1), lambda i: (i, 0)),
        ],
        out_specs=pl.BlockSpec((_BN, D_EXP), lambda i: (i, 0)),
        out_shape=jax.ShapeDtypeStruct((N_NODES, D_EXP), jnp.float32),
    )(dense2, s0, s1, c0, c1)


def _classifier_body(x_ref, w1_ref, b1_ref, w2_ref, b2_ref, out_ref):
    h = jnp.dot(x_ref[...], w1_ref[...], preferred_element_type=jnp.float32)
    h = jnp.maximum(h + b1_ref[...], 0.0)
    z = jnp.dot(h, w2_ref[...], preferred_element_type=jnp.float32)
    out_ref[...] = jax.nn.sigmoid(z + b2_ref[...])


def _classifier(batch_repr, Wc1, bc1, Wc2, bc2):
    return pl.pallas_call(
        _classifier_body,
        in_specs=[
            pl.BlockSpec((N_BATCH, D_EXP), lambda: (0, 0)),
            pl.BlockSpec((D_EXP, 32), lambda: (0, 0)),
            pl.BlockSpec((1, 32), lambda: (0, 0)),
            pl.BlockSpec((32, 1), lambda: (0, 0)),
            pl.BlockSpec((1, 1), lambda: (0, 0)),
        ],
        out_specs=pl.BlockSpec((N_BATCH, 1), lambda: (0, 0)),
        out_shape=jax.ShapeDtypeStruct((N_BATCH, 1), jnp.float32),
    )(batch_repr, Wc1, bc1, Wc2, bc2)


def kernel(node_indices, edge_index, edge_type, node_features, basis1, comp1,
           root1, bias1, basis2, comp2, root2, bias2, Wc1, bc1, Wc2, bc2):
    e3 = jnp.concatenate([edge_index, edge_type[None, :]], axis=0)

    z128 = jnp.zeros((NPAD, D_HID), jnp.float32)
    z64 = jnp.zeros((NPAD, D_EXP), jnp.float32)
    z16 = jnp.zeros((NPAD, 16), jnp.float32)
    ones = jnp.ones((CHUNK, 16), jnp.float32)

    y1, dense1 = _transform1(node_features, basis1, comp1, root1,
                             bias1.reshape(1, D_HID))
    y1_flat = y1.reshape(N_REL * N_NODES, D_HID)
    s1_flat, cnt_flat = _make_edge_kernel(True, D_HID, NBUF1)(
        e3, y1_flat, z128, z16, ones)
    s10 = s1_flat[:N_NODES]
    s11 = s1_flat[NPAD:NPAD + N_NODES]
    c0 = cnt_flat[:N_NODES, :1]
    c1 = cnt_flat[NPAD:NPAD + N_NODES, :1]

    y2, dense2 = _combine2(dense1, s10, s11, c0, c1, basis2, comp2, root2,
                           bias2.reshape(1, D_EXP))
    y2_flat = y2.reshape(N_REL * N_NODES, D_EXP)
    s2_flat = _make_edge_kernel(False, D_EXP, NBUF2)(e3, y2_flat, z64)
    s20 = s2_flat[:N_NODES]
    s21 = s2_flat[NPAD:NPAD + N_NODES]

    out2 = _final_combine(dense2, s20, s21, c0, c1)

    batch_repr = _make_gather_kernel()(out2, node_indices)
    bot_prob = _classifier(batch_repr, Wc1, bc1.reshape(1, 32), Wc2,
                           bc2.reshape(1, 1))
    return (batch_repr, bot_prob)


# speedup vs baseline: 13.9481x; 1.2292x over previous
import jax
import jax.numpy as jnp
from jax import lax
from jax.experimental import pallas as pl
from jax.experimental.pallas import tpu as pltpu
from jax.experimental.pallas import tpu_sc as plsc

N_NODES = 10000
D_IN = 128
D_HID = 128
D_EXP = 64
N_REL = 2
N_EDGES = 320000
N_BATCH = 4096

NC = 2
NS = 16

NPAD = 10112
ROWS_PER_TILE = NPAD // NS
CHUNK = 128
EPT = N_EDGES // NS
NCH = EPT // CHUNK
TAIL = EPT - NCH * CHUNK
NBUF1 = 4
NBUF2 = 6


def _sc_mesh():
    return plsc.VectorSubcoreMesh(
        core_axis_name="c", subcore_axis_name="s", num_cores=NC, num_subcores=NS
    )


def _make_edge_kernel(with_counts, d_half, nbuf):
    out_type = [jax.ShapeDtypeStruct((NC * N_REL * NPAD, d_half), jnp.float32)]
    scratch = [pltpu.VMEM_SHARED((N_REL * NPAD, d_half), jnp.float32)]
    if with_counts:
        out_type.append(jax.ShapeDtypeStruct((NC * NPAD, 16), jnp.float32))
        scratch.append(pltpu.VMEM_SHARED((NPAD, 16), jnp.float32))
    scratch += [pltpu.VMEM((3, CHUNK), jnp.int32) for _ in range(nbuf)]
    scratch += [pltpu.VMEM((CHUNK,), jnp.int32) for _ in range(nbuf)]
    scratch += [pltpu.VMEM((CHUNK,), jnp.int32) for _ in range(nbuf)]
    if with_counts:
        scratch += [pltpu.VMEM((CHUNK,), jnp.int32) for _ in range(nbuf)]
    scratch += [pltpu.VMEM((CHUNK, d_half), jnp.float32)
                for _ in range(nbuf)]
    if with_counts:
        scratch.append(pltpu.VMEM((CHUNK, 16), jnp.float32))
    scratch += [pltpu.VMEM((3, TAIL), jnp.int32),
                pltpu.VMEM((TAIL,), jnp.int32),
                pltpu.VMEM((TAIL,), jnp.int32)]
    if with_counts:
        scratch.append(pltpu.VMEM((TAIL,), jnp.int32))
    n_sem_kinds = 4 if with_counts else 3
    scratch += [pltpu.SemaphoreType.DMA] * (nbuf * n_sem_kinds + 1)

    out_type = tuple(out_type) if with_counts else out_type[0]

    def body(*args):
        if with_counts:
            e3_hbm, y_hbm, z_big, z_small, ones_hbm, s_out, c_out = args[:7]
            rest = list(args[7:])
        else:
            e3_hbm, y_hbm, z_big, s_out = args[:4]
            c_out = z_small = ones_hbm = None
            rest = list(args[4:])
        it = iter(rest)
        acc_sh = next(it)
        cnt_sh = next(it) if with_counts else None
        e3_v = [next(it) for _ in range(nbuf)]
        g_v = [next(it) for _ in range(nbuf)]
        d_v = [next(it) for _ in range(nbuf)]
        q_v = [next(it) for _ in range(nbuf)] if with_counts else None
        rows_v = [next(it) for _ in range(nbuf)]
        ones_v = next(it) if with_counts else None
        e3_t = next(it)
        g_t = next(it)
        d_t = next(it)
        q_t = next(it) if with_counts else None
        sem_i = [next(it) for _ in range(nbuf)]
        sem_g = [next(it) for _ in range(nbuf)]
        sem_s = [next(it) for _ in range(nbuf)]
        sem_c = [next(it) for _ in range(nbuf)] if with_counts else None
        sem_t = next(it)

        c = lax.axis_index("c")
        s = lax.axis_index("s")
        ebase = s * EPT
        lane = lax.iota(jnp.int32, 16)

        a0 = s * (N_REL * ROWS_PER_TILE)
        pltpu.sync_copy(z_big.at[pl.ds(a0, N_REL * ROWS_PER_TILE)],
                        acc_sh.at[pl.ds(a0, N_REL * ROWS_PER_TILE)])
        r0 = s * ROWS_PER_TILE
        if with_counts:
            pltpu.sync_copy(z_small.at[pl.ds(r0, ROWS_PER_TILE)],
                            cnt_sh.at[pl.ds(r0, ROWS_PER_TILE)])
            pltpu.sync_copy(ones_hbm, ones_v)
        plsc.subcore_barrier()

        def idx_src(cj):
            base = ebase + jnp.minimum(cj, NCH - 1) * CHUNK
            return e3_hbm.at[:, pl.ds(base, CHUNK)]

        def issue_idx(cj, b):
            pltpu.async_copy(idx_src(cj), e3_v[b], sem_i[b])

        def wait_idx(b):
            pltpu.make_async_copy(idx_src(0), e3_v[b], sem_i[b]).wait()

        def build(e3r, gr, dr, qr, ngrp):
            for jj in range(ngrp):
                sl = pl.ds(jj * 16, 16)
                sv = e3r[0, sl]
                dv = e3r[1, sl]
                tv = e3r[2, sl]
                gr[sl] = (c * (N_REL * N_NODES)) + tv * N_NODES + sv
                dr[sl] = tv * NPAD + dv
                if qr is not None:
                    trash = N_NODES + ((s * 13 + jj * 16) % 96) + lane
                    qr[sl] = jnp.where(tv == c, dv, trash)

        class _Op:
            def __init__(self, src, dst, sem, add=False):
                self.a = (src, dst, sem)
                self.add = add

            def start(self):
                pltpu.async_copy(*self.a, add=self.add)

            def wait(self):
                pltpu.make_async_copy(*self.a).wait()

        def gather_desc(b):
            return _Op(y_hbm.at[g_v[b]], rows_v[b], sem_g[b])

        def scat_desc(b):
            return _Op(rows_v[b], acc_sh.at[d_v[b]], sem_s[b], add=True)

        def cnt_desc(b):
            return _Op(ones_v, cnt_sh.at[q_v[b]], sem_c[b], add=True)

        for b in range(nbuf):
            issue_idx(b, b)
        for b in range(nbuf):
            wait_idx(b)
            build(e3_v[b], g_v[b], d_v[b], q_v[b] if with_counts else None,
                  CHUNK // 16)
            gather_desc(b).start()
            issue_idx(nbuf + b, b)

        def grp(j, _):
            for b in range(nbuf):
                gather_desc(b).wait()
                scat_desc(b).start()
                if with_counts:
                    cnt_desc(b).start()
            for b in range(nbuf):
                wait_idx(b)
                scat_desc(b).wait()
                if with_counts:
                    cnt_desc(b).wait()
                build(e3_v[b], g_v[b], d_v[b],
                      q_v[b] if with_counts else None, CHUNK // 16)
                gather_desc(b).start()
                issue_idx((j + 2) * nbuf + b, b)
            return 0

        lax.fori_loop(0, NCH // nbuf - 1, grp, 0)

        for b in range(nbuf):
            gather_desc(b).wait()
            scat_desc(b).start()
            if with_counts:
                cnt_desc(b).start()
        for b in range(nbuf):
            wait_idx(b)
            scat_desc(b).wait()
            if with_counts:
                cnt_desc(b).wait()

        rows_t = rows_v[0].at[pl.ds(0, TAIL)]
        pltpu.sync_copy(e3_hbm.at[:, pl.ds(ebase + NCH * CHUNK, TAIL)], e3_t)
        build(e3_t, g_t, d_t, q_t, TAIL // 16)
        pltpu.async_copy(y_hbm.at[g_t], rows_t, sem_t).wait()
        pltpu.sync_copy(rows_t, acc_sh.at[d_t], add=True)
        if with_counts:
            pltpu.sync_copy(ones_v.at[pl.ds(0, TAIL)], cnt_sh.at[q_t],
                            add=True)

        plsc.subcore_barrier()

        pltpu.sync_copy(
            acc_sh.at[pl.ds(a0, N_REL * ROWS_PER_TILE)],
            s_out.at[pl.ds(c * (N_REL * NPAD) + a0, N_REL * ROWS_PER_TILE)])
        if with_counts:
            pltpu.sync_copy(cnt_sh.at[pl.ds(r0, ROWS_PER_TILE)],
                            c_out.at[pl.ds(c * NPAD + r0, ROWS_PER_TILE)])

    return pl.kernel(body, out_type=out_type, mesh=_sc_mesh(),
                     scratch_types=scratch,
                     compiler_params=pltpu.CompilerParams(
                         use_tc_tiling_on_sc=False))


def _gather_kernel_body(table_hbm, idx_hbm, out_hbm, idx_v, rows_v, sem):
    wid = lax.axis_index("s") * NC + lax.axis_index("c")
    per_w = N_BATCH // (NC * NS)
    base = wid * per_w
    pltpu.sync_copy(idx_hbm.at[pl.ds(base, per_w)], idx_v)
    pltpu.async_copy(table_hbm.at[idx_v], rows_v, sem).wait()
    pltpu.sync_copy(rows_v, out_hbm.at[pl.ds(base, per_w)])


def _make_gather_kernel():
    per_w = N_BATCH // (NC * NS)
    return pl.kernel(
        _gather_kernel_body,
        out_type=jax.ShapeDtypeStruct((N_BATCH, D_EXP), jnp.float32),
        mesh=_sc_mesh(),
        scratch_types=[
            pltpu.VMEM((per_w,), jnp.int32),
            pltpu.VMEM((per_w, D_EXP), jnp.float32),
            pltpu.SemaphoreType.DMA,
        ],
        compiler_params=pltpu.CompilerParams(use_tc_tiling_on_sc=False),
    )


_BN = 1000
_H1 = D_HID // 2
_H2 = D_EXP // 2


def _transform1_body(x_ref, basis_ref, comp_ref, root_ref, bias_ref,
                     y_ref, dense_ref):
    x = x_ref[...]
    w0 = comp_ref[0, 0] * basis_ref[0] + comp_ref[0, 1] * basis_ref[1]
    w1 = comp_ref[1, 0] * basis_ref[0] + comp_ref[1, 1] * basis_ref[1]
    y0 = jnp.dot(x, w0, preferred_element_type=jnp.float32)
    y1 = jnp.dot(x, w1, preferred_element_type=jnp.float32)
    y_ref[0, 0] = y0[:, :_H1]
    y_ref[0, 1] = y1[:, :_H1]
    y_ref[1, 0] = y0[:, _H1:]
    y_ref[1, 1] = y1[:, _H1:]
    dense_ref[...] = (
        jnp.dot(x, root_ref[...], preferred_element_type=jnp.float32)
        + bias_ref[...]
    )


def _transform1(x, basis1, comp1, root1, bias1):
    grid = N_NODES // _BN
    return pl.pallas_call(
        _transform1_body,
        grid=(grid,),
        in_specs=[
            pl.BlockSpec((_BN, D_IN), lambda i: (i, 0)),
            pl.BlockSpec((N_REL, D_IN, D_HID), lambda i: (0, 0, 0)),
            pl.BlockSpec(memory_space=pltpu.SMEM),
            pl.BlockSpec((D_IN, D_HID), lambda i: (0, 0)),
            pl.BlockSpec((1, D_HID), lambda i: (0, 0)),
        ],
        out_specs=[
            pl.BlockSpec((NC, N_REL, _BN, _H1), lambda i: (0, 0, i, 0)),
            pl.BlockSpec((_BN, D_HID), lambda i: (i, 0)),
        ],
        out_shape=[
            jax.ShapeDtypeStruct((NC, N_REL, N_NODES, _H1), jnp.float32),
            jax.ShapeDtypeStruct((N_NODES, D_HID), jnp.float32),
        ],
    )(x, basis1, comp1, root1, bias1)


def _combine2_body(dense_ref, s00_ref, s01_ref, s10_ref, s11_ref,
                   c0_ref, c1_ref, basis_ref, comp_ref, root_ref, bias_ref,
                   y_ref, dense2_ref):
    i0 = 1.0 / jnp.maximum(c0_ref[...], 1.0)
    i1 = 1.0 / jnp.maximum(c1_ref[...], 1.0)
    h = dense_ref[...] + jnp.concatenate(
        [s00_ref[...] * i0 + s01_ref[...] * i1,
         s10_ref[...] * i0 + s11_ref[...] * i1], axis=1)
    h = jnp.maximum(h, 0.0)
    w0 = comp_ref[0, 0] * basis_ref[0] + comp_ref[0, 1] * basis_ref[1]
    w1 = comp_ref[1, 0] * basis_ref[0] + comp_ref[1, 1] * basis_ref[1]
    y0 = jnp.dot(h, w0, preferred_element_type=jnp.float32)
    y1 = jnp.dot(h, w1, preferred_element_type=jnp.float32)
    y_ref[0, 0] = y0[:, :_H2]
    y_ref[0, 1] = y1[:, :_H2]
    y_ref[1, 0] = y0[:, _H2:]
    y_ref[1, 1] = y1[:, _H2:]
    dense2_ref[...] = (
        jnp.dot(h, root_ref[...], preferred_element_type=jnp.float32)
        + bias_ref[...]
    )


def _combine2(dense1, s00, s01, s10, s11, c0, c1, basis2, comp2, root2,
              bias2):
    grid = N_NODES // _BN
    half = pl.BlockSpec((_BN, _H1), lambda i: (i, 0))
    cnt = pl.BlockSpec((_BN, 1), lambda i: (i, 0))
    return pl.pallas_call(
        _combine2_body,
        grid=(grid,),
        in_specs=[
            pl.BlockSpec((_BN, D_HID), lambda i: (i, 0)),
            half, half, half, half, cnt, cnt,
            pl.BlockSpec((N_REL, D_HID, D_EXP), lambda i: (0, 0, 0)),
            pl.BlockSpec(memory_space=pltpu.SMEM),
            pl.BlockSpec((D_HID, D_EXP), lambda i: (0, 0)),
            pl.BlockSpec((1, D_EXP), lambda i: (0, 0)),
        ],
        out_specs=[
            pl.BlockSpec((NC, N_REL, _BN, _H2), lambda i: (0, 0, i, 0)),
            pl.BlockSpec((_BN, D_EXP), lambda i: (i, 0)),
        ],
        out_shape=[
            jax.ShapeDtypeStruct((NC, N_REL, N_NODES, _H2), jnp.float32),
            jax.ShapeDtypeStruct((N_NODES, D_EXP), jnp.float32),
        ],
    )(dense1, s00, s01, s10, s11, c0, c1, basis2, comp2, root2, bias2)


def _final_combine_body(dense_ref, s00_ref, s01_ref, s10_ref, s11_ref,
                        c0_ref, c1_ref, out_ref):
    i0 = 1.0 / jnp.maximum(c0_ref[...], 1.0)
    i1 = 1.0 / jnp.maximum(c1_ref[...], 1.0)
    out_ref[...] = dense_ref[...] + jnp.concatenate(
        [s00_ref[...] * i0 + s01_ref[...] * i1,
         s10_ref[...] * i0 + s11_ref[...] * i1], axis=1)


def _final_combine(dense2, s00, s01, s10, s11, c0, c1):
    grid = N_NODES // _BN
    half = pl.BlockSpec((_BN, _H2), lambda i: (i, 0))
    cnt = pl.BlockSpec((_BN, 1), lambda i: (i, 0))
    return pl.pallas_call(
        _final_combine_body,
        grid=(grid,),
        in_specs=[pl.BlockSpec((_BN, D_EXP), lambda i: (i, 0)),
                  half, half, half, half, cnt, cnt],
        out_specs=pl.BlockSpec((_BN, D_EXP), lambda i: (i, 0)),
        out_shape=jax.ShapeDtypeStruct((N_NODES, D_EXP), jnp.float32),
    )(dense2, s00, s01, s10, s11, c0, c1)


def _classifier_body(x_ref, w1_ref, b1_ref, w2_ref, b2_ref, out_ref):
    h = jnp.dot(x_ref[...], w1_ref[...], preferred_element_type=jnp.float32)
    h = jnp.maximum(h + b1_ref[...], 0.0)
    z = jnp.dot(h, w2_ref[...], preferred_element_type=jnp.float32)
    out_ref[...] = jax.nn.sigmoid(z + b2_ref[...])


def _classifier(batch_repr, Wc1, bc1, Wc2, bc2):
    return pl.pallas_call(
        _classifier_body,
        in_specs=[
            pl.BlockSpec((N_BATCH, D_EXP), lambda: (0, 0)),
            pl.BlockSpec((D_EXP, 32), lambda: (0, 0)),
            pl.BlockSpec((1, 32), lambda: (0, 0)),
            pl.BlockSpec((32, 1), lambda: (0, 0)),
            pl.BlockSpec((1, 1), lambda: (0, 0)),
        ],
        out_specs=pl.BlockSpec((N_BATCH, 1), lambda: (0, 0)),
        out_shape=jax.ShapeDtypeStruct((N_BATCH, 1), jnp.float32),
    )(batch_repr, Wc1, bc1, Wc2, bc2)


def kernel(node_indices, edge_index, edge_type, node_features, basis1, comp1,
           root1, bias1, basis2, comp2, root2, bias2, Wc1, bc1, Wc2, bc2):
    e3 = jnp.concatenate([edge_index, edge_type[None, :]], axis=0)

    zb1 = jnp.zeros((N_REL * NPAD, _H1), jnp.float32)
    zb2 = jnp.zeros((N_REL * NPAD, _H2), jnp.float32)
    z16 = jnp.zeros((NPAD, 16), jnp.float32)
    ones = jnp.ones((CHUNK, 16), jnp.float32)

    y1, dense1 = _transform1(node_features, basis1, comp1, root1,
                             bias1.reshape(1, D_HID))
    y1_flat = y1.reshape(NC * N_REL * N_NODES, _H1)
    s1_flat, cnt_flat = _make_edge_kernel(True, _H1, NBUF1)(
        e3, y1_flat, zb1, z16, ones)
    s1 = [s1_flat[k * NPAD:k * NPAD + N_NODES] for k in range(4)]
    c0 = cnt_flat[:N_NODES, :1]
    c1 = cnt_flat[NPAD:NPAD + N_NODES, :1]

    y2, dense2 = _combine2(dense1, s1[0], s1[1], s1[2], s1[3], c0, c1,
                           basis2, comp2, root2, bias2.reshape(1, D_EXP))
    y2_flat = y2.reshape(NC * N_REL * N_NODES, _H2)
    s2_flat = _make_edge_kernel(False, _H2, NBUF2)(e3, y2_flat, zb2)
    s2 = [s2_flat[k * NPAD:k * NPAD + N_NODES] for k in range(4)]

    out2 = _final_combine(dense2, s2[0], s2[1], s2[2], s2[3], c0, c1)

    batch_repr = _make_gather_kernel()(out2, node_indices)
    bot_prob = _classifier(batch_repr, Wc1, bc1.reshape(1, 32), Wc2,
                           bc2.reshape(1, 1))
    return (batch_repr, bot_prob)


# speedup vs baseline: 15.0029x; 1.0756x over previous
import jax
import jax.numpy as jnp
from jax import lax
from jax.experimental import pallas as pl
from jax.experimental.pallas import tpu as pltpu
from jax.experimental.pallas import tpu_sc as plsc

N_NODES = 10000
D_IN = 128
D_HID = 128
D_EXP = 64
N_REL = 2
N_EDGES = 320000
N_BATCH = 4096

NC = 2
NS = 16

NPAD = 10112
ROWS_PER_TILE = NPAD // NS
CHUNK = 128
EPT = N_EDGES // NS
NCH = EPT // CHUNK
TAIL = EPT - NCH * CHUNK
NBUF1 = 4
NBUF2 = 6


def _sc_mesh():
    return plsc.VectorSubcoreMesh(
        core_axis_name="c", subcore_axis_name="s", num_cores=NC, num_subcores=NS
    )


_BPT = N_BATCH // NS


def _make_edge_kernel(with_counts, d_half, nbuf, final=False):
    out_type = [jax.ShapeDtypeStruct((NC * N_REL * NPAD, d_half), jnp.float32)]
    scratch = [pltpu.VMEM_SHARED((N_REL * NPAD, d_half), jnp.float32)]
    if final:
        out_type.append(
            jax.ShapeDtypeStruct((NC * N_BATCH, d_half), jnp.float32))
    if with_counts:
        out_type.append(jax.ShapeDtypeStruct((NC * NPAD, 16), jnp.float32))
        scratch.append(pltpu.VMEM_SHARED((NPAD, 16), jnp.float32))
    scratch += [pltpu.VMEM((3, CHUNK), jnp.int32) for _ in range(nbuf)]
    scratch += [pltpu.VMEM((CHUNK,), jnp.int32) for _ in range(nbuf)]
    scratch += [pltpu.VMEM((CHUNK,), jnp.int32) for _ in range(nbuf)]
    if with_counts:
        scratch += [pltpu.VMEM((CHUNK,), jnp.int32) for _ in range(nbuf)]
    scratch += [pltpu.VMEM((CHUNK, d_half), jnp.float32)
                for _ in range(nbuf)]
    if with_counts:
        scratch.append(pltpu.VMEM((CHUNK, 16), jnp.float32))
    scratch += [pltpu.VMEM((3, TAIL), jnp.int32),
                pltpu.VMEM((TAIL,), jnp.int32),
                pltpu.VMEM((TAIL,), jnp.int32)]
    if with_counts:
        scratch.append(pltpu.VMEM((TAIL,), jnp.int32))
    if final:
        scratch += [pltpu.VMEM((2, CHUNK), jnp.int32)]
        scratch += [pltpu.VMEM((2, CHUNK), jnp.int32) for _ in range(5)]
        scratch += [pltpu.VMEM((_BPT, d_half), jnp.float32)
                    for _ in range(3)]
        scratch += [pltpu.VMEM((_BPT, 16), jnp.float32) for _ in range(2)]
        scratch += [pltpu.VMEM((_BPT, d_half), jnp.float32)]
    n_sem_kinds = 4 if with_counts else 3
    scratch += [pltpu.SemaphoreType.DMA] * (nbuf * n_sem_kinds + 1)

    out_type = tuple(out_type) if (with_counts or final) else out_type[0]

    def body(*args):
        if with_counts:
            e3_hbm, y_hbm, z_big, z_small, ones_hbm, s_out, c_out = args[:7]
            rest = list(args[7:])
            idx_hbm = d2_hbm = cnt_hbm = p_out = None
        elif final:
            e3_hbm, y_hbm, z_big, idx_hbm, d2_hbm, cnt_hbm = args[:6]
            s_out, p_out = args[6:8]
            c_out = z_small = ones_hbm = None
            rest = list(args[8:])
        else:
            e3_hbm, y_hbm, z_big, s_out = args[:4]
            c_out = z_small = ones_hbm = None
            idx_hbm = d2_hbm = cnt_hbm = p_out = None
            rest = list(args[4:])
        it = iter(rest)
        acc_sh = next(it)
        cnt_sh = next(it) if with_counts else None
        e3_v = [next(it) for _ in range(nbuf)]
        g_v = [next(it) for _ in range(nbuf)]
        d_v = [next(it) for _ in range(nbuf)]
        q_v = [next(it) for _ in range(nbuf)] if with_counts else None
        rows_v = [next(it) for _ in range(nbuf)]
        ones_v = next(it) if with_counts else None
        e3_t = next(it)
        g_t = next(it)
        d_t = next(it)
        q_t = next(it) if with_counts else None
        if final:
            bidx_v = next(it)
            bg = [next(it) for _ in range(5)]
            dD = next(it)
            s0r = next(it)
            s1r = next(it)
            c0r = next(it)
            c1r = next(it)
            br = next(it)
        sem_i = [next(it) for _ in range(nbuf)]
        sem_g = [next(it) for _ in range(nbuf)]
        sem_s = [next(it) for _ in range(nbuf)]
        sem_c = [next(it) for _ in range(nbuf)] if with_counts else None
        sem_t = next(it)

        c = lax.axis_index("c")
        s = lax.axis_index("s")
        ebase = s * EPT
        lane = lax.iota(jnp.int32, 16)

        a0 = s * (N_REL * ROWS_PER_TILE)
        pltpu.sync_copy(z_big.at[pl.ds(a0, N_REL * ROWS_PER_TILE)],
                        acc_sh.at[pl.ds(a0, N_REL * ROWS_PER_TILE)])
        r0 = s * ROWS_PER_TILE
        if with_counts:
            pltpu.sync_copy(z_small.at[pl.ds(r0, ROWS_PER_TILE)],
                            cnt_sh.at[pl.ds(r0, ROWS_PER_TILE)])
            pltpu.sync_copy(ones_hbm, ones_v)
        plsc.subcore_barrier()

        def idx_src(cj):
            base = ebase + jnp.minimum(cj, NCH - 1) * CHUNK
            return e3_hbm.at[:, pl.ds(base, CHUNK)]

        def issue_idx(cj, b):
            pltpu.async_copy(idx_src(cj), e3_v[b], sem_i[b])

        def wait_idx(b):
            pltpu.make_async_copy(idx_src(0), e3_v[b], sem_i[b]).wait()

        def build(e3r, gr, dr, qr, ngrp):
            for jj in range(ngrp):
                sl = pl.ds(jj * 16, 16)
                sv = e3r[0, sl]
                dv = e3r[1, sl]
                tv = e3r[2, sl]
                gr[sl] = (c * (N_REL * N_NODES)) + tv * N_NODES + sv
                dr[sl] = tv * NPAD + dv
                if qr is not None:
                    trash = N_NODES + ((s * 13 + jj * 16) % 96) + lane
                    qr[sl] = jnp.where(tv == c, dv, trash)

        class _Op:
            def __init__(self, src, dst, sem, add=False):
                self.a = (src, dst, sem)
                self.add = add

            def start(self):
                pltpu.async_copy(*self.a, add=self.add)

            def wait(self):
                pltpu.make_async_copy(*self.a).wait()

        def gather_desc(b):
            return _Op(y_hbm.at[g_v[b]], rows_v[b], sem_g[b])

        def scat_desc(b):
            return _Op(rows_v[b], acc_sh.at[d_v[b]], sem_s[b], add=True)

        def cnt_desc(b):
            return _Op(ones_v, cnt_sh.at[q_v[b]], sem_c[b], add=True)

        for b in range(nbuf):
            issue_idx(b, b)
        for b in range(nbuf):
            wait_idx(b)
            build(e3_v[b], g_v[b], d_v[b], q_v[b] if with_counts else None,
                  CHUNK // 16)
            gather_desc(b).start()
            issue_idx(nbuf + b, b)

        def grp(j, _):
            for b in range(nbuf):
                gather_desc(b).wait()
                scat_desc(b).start()
                if with_counts:
                    cnt_desc(b).start()
            for b in range(nbuf):
                wait_idx(b)
                scat_desc(b).wait()
                if with_counts:
                    cnt_desc(b).wait()
                build(e3_v[b], g_v[b], d_v[b],
                      q_v[b] if with_counts else None, CHUNK // 16)
                gather_desc(b).start()
                issue_idx((j + 2) * nbuf + b, b)
            return 0

        lax.fori_loop(0, NCH // nbuf - 1, grp, 0)

        for b in range(nbuf):
            gather_desc(b).wait()
            scat_desc(b).start()
            if with_counts:
                cnt_desc(b).start()
        for b in range(nbuf):
            wait_idx(b)
            scat_desc(b).wait()
            if with_counts:
                cnt_desc(b).wait()

        rows_t = rows_v[0].at[pl.ds(0, TAIL)]
        pltpu.sync_copy(e3_hbm.at[:, pl.ds(ebase + NCH * CHUNK, TAIL)], e3_t)
        build(e3_t, g_t, d_t, q_t, TAIL // 16)
        pltpu.async_copy(y_hbm.at[g_t], rows_t, sem_t).wait()
        pltpu.sync_copy(rows_t, acc_sh.at[d_t], add=True)
        if with_counts:
            pltpu.sync_copy(ones_v.at[pl.ds(0, TAIL)], cnt_sh.at[q_t],
                            add=True)

        plsc.subcore_barrier()

        pltpu.sync_copy(
            acc_sh.at[pl.ds(a0, N_REL * ROWS_PER_TILE)],
            s_out.at[pl.ds(c * (N_REL * NPAD) + a0, N_REL * ROWS_PER_TILE)])
        if with_counts:
            pltpu.sync_copy(cnt_sh.at[pl.ds(r0, ROWS_PER_TILE)],
                            c_out.at[pl.ds(c * NPAD + r0, ROWS_PER_TILE)])

        if final:
            plsc.subcore_barrier()
            bb = s * _BPT
            for k in range(2):
                pltpu.sync_copy(idx_hbm.at[pl.ds(bb + k * CHUNK, CHUNK)],
                                bidx_v.at[k])
            for k in range(2):
                for jj in range(CHUNK // 16):
                    sl = pl.ds(jj * 16, 16)
                    iv = bidx_v[k, sl]
                    bg[0][k, sl] = c * N_NODES + iv
                    bg[1][k, sl] = c * (N_REL * NPAD) + iv
                    bg[2][k, sl] = c * (N_REL * NPAD) + NPAD + iv
                    bg[3][k, sl] = iv
                    bg[4][k, sl] = NPAD + iv
            gat = []
            for k in range(2):
                dsl = pl.ds(k * CHUNK, CHUNK)
                gat += [
                    _Op(d2_hbm.at[bg[0].at[k]], dD.at[dsl], sem_t),
                    _Op(s_out.at[bg[1].at[k]], s0r.at[dsl], sem_t),
                    _Op(s_out.at[bg[2].at[k]], s1r.at[dsl], sem_t),
                    _Op(cnt_hbm.at[bg[3].at[k]], c0r.at[dsl], sem_t),
                    _Op(cnt_hbm.at[bg[4].at[k]], c1r.at[dsl], sem_t),
                ]
            for op in gat:
                op.start()
            for op in gat:
                op.wait()

            def comb(i, _):
                inv0 = 1.0 / jnp.maximum(c0r[i, pl.ds(0, 16)], 1.0)
                inv1 = 1.0 / jnp.maximum(c1r[i, pl.ds(0, 16)], 1.0)
                for k in range(d_half // 16):
                    sl = pl.ds(k * 16, 16)
                    br[i, sl] = (dD[i, sl] + s0r[i, sl] * inv0
                                 + s1r[i, sl] * inv1)
                return 0

            lax.fori_loop(0, _BPT, comb, 0)
            pltpu.sync_copy(br, p_out.at[pl.ds(c * N_BATCH + bb, _BPT)])

    return pl.kernel(body, out_type=out_type, mesh=_sc_mesh(),
                     scratch_types=scratch,
                     compiler_params=pltpu.CompilerParams(
                         use_tc_tiling_on_sc=False))


def _gather_kernel_body(table_hbm, idx_hbm, out_hbm, idx_v, rows_v, sem):
    wid = lax.axis_index("s") * NC + lax.axis_index("c")
    per_w = N_BATCH // (NC * NS)
    base = wid * per_w
    pltpu.sync_copy(idx_hbm.at[pl.ds(base, per_w)], idx_v)
    pltpu.async_copy(table_hbm.at[idx_v], rows_v, sem).wait()
    pltpu.sync_copy(rows_v, out_hbm.at[pl.ds(base, per_w)])


def _make_gather_kernel():
    per_w = N_BATCH // (NC * NS)
    return pl.kernel(
        _gather_kernel_body,
        out_type=jax.ShapeDtypeStruct((N_BATCH, D_EXP), jnp.float32),
        mesh=_sc_mesh(),
        scratch_types=[
            pltpu.VMEM((per_w,), jnp.int32),
            pltpu.VMEM((per_w, D_EXP), jnp.float32),
            pltpu.SemaphoreType.DMA,
        ],
        compiler_params=pltpu.CompilerParams(use_tc_tiling_on_sc=False),
    )


_BN = 1000
_H1 = D_HID // 2
_H2 = D_EXP // 2


def _transform1_body(x_ref, basis_ref, comp_ref, root_ref, bias_ref,
                     y_ref, dense_ref):
    x = x_ref[...]
    w0 = comp_ref[0, 0] * basis_ref[0] + comp_ref[0, 1] * basis_ref[1]
    w1 = comp_ref[1, 0] * basis_ref[0] + comp_ref[1, 1] * basis_ref[1]
    y0 = jnp.dot(x, w0, preferred_element_type=jnp.float32)
    y1 = jnp.dot(x, w1, preferred_element_type=jnp.float32)
    y_ref[0, 0] = y0[:, :_H1]
    y_ref[0, 1] = y1[:, :_H1]
    y_ref[1, 0] = y0[:, _H1:]
    y_ref[1, 1] = y1[:, _H1:]
    dense_ref[...] = (
        jnp.dot(x, root_ref[...], preferred_element_type=jnp.float32)
        + bias_ref[...]
    )


def _transform1(x, basis1, comp1, root1, bias1):
    grid = N_NODES // _BN
    return pl.pallas_call(
        _transform1_body,
        grid=(grid,),
        in_specs=[
            pl.BlockSpec((_BN, D_IN), lambda i: (i, 0)),
            pl.BlockSpec((N_REL, D_IN, D_HID), lambda i: (0, 0, 0)),
            pl.BlockSpec(memory_space=pltpu.SMEM),
            pl.BlockSpec((D_IN, D_HID), lambda i: (0, 0)),
            pl.BlockSpec((1, D_HID), lambda i: (0, 0)),
        ],
        out_specs=[
            pl.BlockSpec((NC, N_REL, _BN, _H1), lambda i: (0, 0, i, 0)),
            pl.BlockSpec((_BN, D_HID), lambda i: (i, 0)),
        ],
        out_shape=[
            jax.ShapeDtypeStruct((NC, N_REL, N_NODES, _H1), jnp.float32),
            jax.ShapeDtypeStruct((N_NODES, D_HID), jnp.float32),
        ],
    )(x, basis1, comp1, root1, bias1)


def _combine2_body(dense_ref, s00_ref, s01_ref, s10_ref, s11_ref,
                   c0_ref, c1_ref, basis_ref, comp_ref, root_ref, bias_ref,
                   y_ref, dense2_ref):
    i0 = 1.0 / jnp.maximum(c0_ref[...], 1.0)
    i1 = 1.0 / jnp.maximum(c1_ref[...], 1.0)
    h = dense_ref[...] + jnp.concatenate(
        [s00_ref[...] * i0 + s01_ref[...] * i1,
         s10_ref[...] * i0 + s11_ref[...] * i1], axis=1)
    h = jnp.maximum(h, 0.0)
    w0 = comp_ref[0, 0] * basis_ref[0] + comp_ref[0, 1] * basis_ref[1]
    w1 = comp_ref[1, 0] * basis_ref[0] + comp_ref[1, 1] * basis_ref[1]
    y0 = jnp.dot(h, w0, preferred_element_type=jnp.float32)
    y1 = jnp.dot(h, w1, preferred_element_type=jnp.float32)
    y_ref[0, 0] = y0[:, :_H2]
    y_ref[0, 1] = y1[:, :_H2]
    y_ref[1, 0] = y0[:, _H2:]
    y_ref[1, 1] = y1[:, _H2:]
    d2 = (jnp.dot(h, root_ref[...], preferred_element_type=jnp.float32)
          + bias_ref[...])
    dense2_ref[0] = d2[:, :_H2]
    dense2_ref[1] = d2[:, _H2:]


def _combine2(dense1, s00, s01, s10, s11, c0, c1, basis2, comp2, root2,
              bias2):
    grid = N_NODES // _BN
    half = pl.BlockSpec((_BN, _H1), lambda i: (i, 0))
    cnt = pl.BlockSpec((_BN, 1), lambda i: (i, 0))
    return pl.pallas_call(
        _combine2_body,
        grid=(grid,),
        in_specs=[
            pl.BlockSpec((_BN, D_HID), lambda i: (i, 0)),
            half, half, half, half, cnt, cnt,
            pl.BlockSpec((N_REL, D_HID, D_EXP), lambda i: (0, 0, 0)),
            pl.BlockSpec(memory_space=pltpu.SMEM),
            pl.BlockSpec((D_HID, D_EXP), lambda i: (0, 0)),
            pl.BlockSpec((1, D_EXP), lambda i: (0, 0)),
        ],
        out_specs=[
            pl.BlockSpec((NC, N_REL, _BN, _H2), lambda i: (0, 0, i, 0)),
            pl.BlockSpec((NC, _BN, _H2), lambda i: (0, i, 0)),
        ],
        out_shape=[
            jax.ShapeDtypeStruct((NC, N_REL, N_NODES, _H2), jnp.float32),
            jax.ShapeDtypeStruct((NC, N_NODES, _H2), jnp.float32),
        ],
    )(dense1, s00, s01, s10, s11, c0, c1, basis2, comp2, root2, bias2)


def _classifier_body(p0_ref, p1_ref, w1_ref, b1_ref, w2_ref, b2_ref,
                     br_ref, out_ref):
    x = jnp.concatenate([p0_ref[...], p1_ref[...]], axis=1)
    br_ref[...] = x
    h = jnp.dot(x, w1_ref[...], preferred_element_type=jnp.float32)
    h = jnp.maximum(h + b1_ref[...], 0.0)
    z = jnp.dot(h, w2_ref[...], preferred_element_type=jnp.float32)
    out_ref[...] = jax.nn.sigmoid(z + b2_ref[...])


def _classifier(p0, p1, Wc1, bc1, Wc2, bc2):
    return pl.pallas_call(
        _classifier_body,
        in_specs=[
            pl.BlockSpec((N_BATCH, _H2), lambda: (0, 0)),
            pl.BlockSpec((N_BATCH, _H2), lambda: (0, 0)),
            pl.BlockSpec((D_EXP, 32), lambda: (0, 0)),
            pl.BlockSpec((1, 32), lambda: (0, 0)),
            pl.BlockSpec((32, 1), lambda: (0, 0)),
            pl.BlockSpec((1, 1), lambda: (0, 0)),
        ],
        out_specs=[
            pl.BlockSpec((N_BATCH, D_EXP), lambda: (0, 0)),
            pl.BlockSpec((N_BATCH, 1), lambda: (0, 0)),
        ],
        out_shape=[
            jax.ShapeDtypeStruct((N_BATCH, D_EXP), jnp.float32),
            jax.ShapeDtypeStruct((N_BATCH, 1), jnp.float32),
        ],
    )(p0, p1, Wc1, bc1, Wc2, bc2)


def kernel(node_indices, edge_index, edge_type, node_features, basis1, comp1,
           root1, bias1, basis2, comp2, root2, bias2, Wc1, bc1, Wc2, bc2):
    e3 = jnp.concatenate([edge_index, edge_type[None, :]], axis=0)

    zb1 = jnp.zeros((N_REL * NPAD, _H1), jnp.float32)
    zb2 = jnp.zeros((N_REL * NPAD, _H2), jnp.float32)
    z16 = jnp.zeros((NPAD, 16), jnp.float32)
    ones = jnp.ones((CHUNK, 16), jnp.float32)

    y1, dense1 = _transform1(node_features, basis1, comp1, root1,
                             bias1.reshape(1, D_HID))
    y1_flat = y1.reshape(NC * N_REL * N_NODES, _H1)
    s1_flat, cnt_flat = _make_edge_kernel(True, _H1, NBUF1)(
        e3, y1_flat, zb1, z16, ones)
    s1 = [s1_flat[k * NPAD:k * NPAD + N_NODES] for k in range(4)]
    c0 = cnt_flat[:N_NODES, :1]
    c1 = cnt_flat[NPAD:NPAD + N_NODES, :1]

    y2, dense2c = _combine2(dense1, s1[0], s1[1], s1[2], s1[3], c0, c1,
                            basis2, comp2, root2, bias2.reshape(1, D_EXP))
    y2_flat = y2.reshape(NC * N_REL * N_NODES, _H2)
    d2_flat = dense2c.reshape(NC * N_NODES, _H2)
    _, p_flat = _make_edge_kernel(False, _H2, NBUF2, final=True)(
        e3, y2_flat, zb2, node_indices, d2_flat, cnt_flat)

    batch_repr, bot_prob = _classifier(
        p_flat[:N_BATCH], p_flat[N_BATCH:], Wc1, bc1.reshape(1, 32), Wc2,
        bc2.reshape(1, 1))
    return (batch_repr, bot_prob)


# speedup vs baseline: 15.4569x; 1.0303x over previous
import jax
import jax.numpy as jnp
from jax import lax
from jax.experimental import pallas as pl
from jax.experimental.pallas import tpu as pltpu
from jax.experimental.pallas import tpu_sc as plsc

N_NODES = 10000
D_IN = 128
D_HID = 128
D_EXP = 64
N_REL = 2
N_EDGES = 320000
N_BATCH = 4096

NC = 2
NS = 16

NPAD = 10112
ROWS_PER_TILE = NPAD // NS
CHUNK = 128
EPT = N_EDGES // NS
NCH = EPT // CHUNK
TAIL = EPT - NCH * CHUNK
NBUF1 = 4
NBUF2 = 6


def _sc_mesh():
    return plsc.VectorSubcoreMesh(
        core_axis_name="c", subcore_axis_name="s", num_cores=NC, num_subcores=NS
    )


_BPT = N_BATCH // NS


def _make_edge_kernel(with_counts, d_half, nbuf, final=False):
    s_w = 128 if with_counts else d_half
    out_type = [jax.ShapeDtypeStruct((NC * N_REL * NPAD, s_w), jnp.float32)]
    scratch = [pltpu.VMEM_SHARED((N_REL * NPAD, d_half), jnp.float32)]
    if final:
        out_type.append(
            jax.ShapeDtypeStruct((NC * N_BATCH, d_half), jnp.float32))
    if with_counts:
        out_type.append(jax.ShapeDtypeStruct((NC * NPAD, 128), jnp.float32))
        out_type.append(jax.ShapeDtypeStruct((NC * NPAD, 16), jnp.float32))
        scratch.append(pltpu.VMEM_SHARED((NPAD, 16), jnp.float32))
    scratch += [pltpu.VMEM((3, CHUNK), jnp.int32) for _ in range(nbuf)]
    scratch += [pltpu.VMEM((CHUNK,), jnp.int32) for _ in range(nbuf)]
    scratch += [pltpu.VMEM((CHUNK,), jnp.int32) for _ in range(nbuf)]
    if with_counts:
        scratch += [pltpu.VMEM((CHUNK,), jnp.int32) for _ in range(nbuf)]
    scratch += [pltpu.VMEM((CHUNK, d_half), jnp.float32)
                for _ in range(nbuf)]
    if with_counts:
        scratch.append(pltpu.VMEM((CHUNK, 16), jnp.float32))
    scratch += [pltpu.VMEM((3, TAIL), jnp.int32),
                pltpu.VMEM((TAIL,), jnp.int32),
                pltpu.VMEM((TAIL,), jnp.int32)]
    if with_counts:
        scratch.append(pltpu.VMEM((TAIL,), jnp.int32))
    if final:
        scratch += [pltpu.VMEM((2, CHUNK), jnp.int32)]
        scratch += [pltpu.VMEM((2, CHUNK), jnp.int32) for _ in range(5)]
        scratch += [pltpu.VMEM((_BPT, d_half), jnp.float32)
                    for _ in range(3)]
        scratch += [pltpu.VMEM((_BPT, 16), jnp.float32) for _ in range(2)]
        scratch += [pltpu.VMEM((_BPT, d_half), jnp.float32)]
    n_sem_kinds = 4 if with_counts else 3
    scratch += [pltpu.SemaphoreType.DMA] * (nbuf * n_sem_kinds + 1)

    out_type = tuple(out_type) if (with_counts or final) else out_type[0]

    def body(*args):
        if with_counts:
            e3_hbm, y_hbm, z_big, z_small, ones_hbm = args[:5]
            s_out, c_out_w, c_out = args[5:8]
            rest = list(args[8:])
            idx_hbm = d2_hbm = cnt_hbm = p_out = None
        elif final:
            e3_hbm, y_hbm, z_big, idx_hbm, d2_hbm, cnt_hbm = args[:6]
            s_out, p_out = args[6:8]
            c_out = c_out_w = z_small = ones_hbm = None
            rest = list(args[8:])
        else:
            e3_hbm, y_hbm, z_big, s_out = args[:4]
            c_out = c_out_w = z_small = ones_hbm = None
            idx_hbm = d2_hbm = cnt_hbm = p_out = None
            rest = list(args[4:])
        it = iter(rest)
        acc_sh = next(it)
        cnt_sh = next(it) if with_counts else None
        e3_v = [next(it) for _ in range(nbuf)]
        g_v = [next(it) for _ in range(nbuf)]
        d_v = [next(it) for _ in range(nbuf)]
        q_v = [next(it) for _ in range(nbuf)] if with_counts else None
        rows_v = [next(it) for _ in range(nbuf)]
        ones_v = next(it) if with_counts else None
        e3_t = next(it)
        g_t = next(it)
        d_t = next(it)
        q_t = next(it) if with_counts else None
        if final:
            bidx_v = next(it)
            bg = [next(it) for _ in range(5)]
            dD = next(it)
            s0r = next(it)
            s1r = next(it)
            c0r = next(it)
            c1r = next(it)
            br = next(it)
        sem_i = [next(it) for _ in range(nbuf)]
        sem_g = [next(it) for _ in range(nbuf)]
        sem_s = [next(it) for _ in range(nbuf)]
        sem_c = [next(it) for _ in range(nbuf)] if with_counts else None
        sem_t = next(it)

        c = lax.axis_index("c")
        s = lax.axis_index("s")
        ebase = s * EPT
        lane = lax.iota(jnp.int32, 16)

        a0 = s * (N_REL * ROWS_PER_TILE)
        zr = N_REL * ROWS_PER_TILE // 4
        for k in range(4):
            pltpu.sync_copy(z_big, acc_sh.at[pl.ds(a0 + k * zr, zr)])
        r0 = s * ROWS_PER_TILE
        if with_counts:
            pltpu.sync_copy(z_small, cnt_sh.at[pl.ds(r0, ROWS_PER_TILE)])
            pltpu.sync_copy(ones_hbm, ones_v)
        plsc.subcore_barrier()

        def idx_src(cj):
            base = ebase + jnp.minimum(cj, NCH - 1) * CHUNK
            return e3_hbm.at[:, pl.ds(base, CHUNK)]

        def issue_idx(cj, b):
            pltpu.async_copy(idx_src(cj), e3_v[b], sem_i[b])

        def wait_idx(b):
            pltpu.make_async_copy(idx_src(0), e3_v[b], sem_i[b]).wait()

        def build(e3r, gr, dr, qr, ngrp):
            for jj in range(ngrp):
                sl = pl.ds(jj * 16, 16)
                sv = e3r[0, sl]
                dv = e3r[1, sl]
                tv = e3r[2, sl]
                gr[sl] = (c * (N_REL * N_NODES)) + tv * N_NODES + sv
                dr[sl] = tv * NPAD + dv
                if qr is not None:
                    trash = N_NODES + ((s * 13 + jj * 16) % 96) + lane
                    qr[sl] = jnp.where(tv == c, dv, trash)

        class _Op:
            def __init__(self, src, dst, sem, add=False):
                self.a = (src, dst, sem)
                self.add = add

            def start(self):
                pltpu.async_copy(*self.a, add=self.add)

            def wait(self):
                pltpu.make_async_copy(*self.a).wait()

        def gather_desc(b):
            return _Op(y_hbm.at[g_v[b]], rows_v[b], sem_g[b])

        def scat_desc(b):
            return _Op(rows_v[b], acc_sh.at[d_v[b]], sem_s[b], add=True)

        def cnt_desc(b):
            return _Op(ones_v, cnt_sh.at[q_v[b]], sem_c[b], add=True)

        for b in range(nbuf):
            issue_idx(b, b)
        for b in range(nbuf):
            wait_idx(b)
            build(e3_v[b], g_v[b], d_v[b], q_v[b] if with_counts else None,
                  CHUNK // 16)
            gather_desc(b).start()
            issue_idx(nbuf + b, b)

        def grp(j, _):
            for b in range(nbuf):
                gather_desc(b).wait()
                scat_desc(b).start()
                if with_counts:
                    cnt_desc(b).start()
            for b in range(nbuf):
                wait_idx(b)
                scat_desc(b).wait()
                if with_counts:
                    cnt_desc(b).wait()
                build(e3_v[b], g_v[b], d_v[b],
                      q_v[b] if with_counts else None, CHUNK // 16)
                gather_desc(b).start()
                issue_idx((j + 2) * nbuf + b, b)
            return 0

        lax.fori_loop(0, NCH // nbuf - 1, grp, 0)

        for b in range(nbuf):
            gather_desc(b).wait()
            scat_desc(b).start()
            if with_counts:
                cnt_desc(b).start()
        for b in range(nbuf):
            wait_idx(b)
            scat_desc(b).wait()
            if with_counts:
                cnt_desc(b).wait()

        rows_t = rows_v[0].at[pl.ds(0, TAIL)]
        pltpu.sync_copy(e3_hbm.at[:, pl.ds(ebase + NCH * CHUNK, TAIL)], e3_t)
        build(e3_t, g_t, d_t, q_t, TAIL // 16)
        pltpu.async_copy(y_hbm.at[g_t], rows_t, sem_t).wait()
        pltpu.sync_copy(rows_t, acc_sh.at[d_t], add=True)
        if with_counts:
            pltpu.sync_copy(ones_v.at[pl.ds(0, TAIL)], cnt_sh.at[q_t],
                            add=True)

        plsc.subcore_barrier()

        so0 = c * (N_REL * NPAD) + a0
        if with_counts:
            pltpu.sync_copy(
                acc_sh.at[pl.ds(a0, N_REL * ROWS_PER_TILE)],
                s_out.at[pl.ds(so0, N_REL * ROWS_PER_TILE), pl.ds(0, d_half)])
            co0 = c * NPAD + r0
            pltpu.sync_copy(cnt_sh.at[pl.ds(r0, ROWS_PER_TILE)],
                            c_out_w.at[pl.ds(co0, ROWS_PER_TILE),
                                       pl.ds(0, 16)])
            pltpu.sync_copy(cnt_sh.at[pl.ds(r0, ROWS_PER_TILE)],
                            c_out.at[pl.ds(co0, ROWS_PER_TILE)])
        else:
            pltpu.sync_copy(
                acc_sh.at[pl.ds(a0, N_REL * ROWS_PER_TILE)],
                s_out.at[pl.ds(so0, N_REL * ROWS_PER_TILE)])

        if final:
            plsc.subcore_barrier()
            bb = s * _BPT
            for k in range(2):
                pltpu.sync_copy(idx_hbm.at[pl.ds(bb + k * CHUNK, CHUNK)],
                                bidx_v.at[k])
            for k in range(2):
                for jj in range(CHUNK // 16):
                    sl = pl.ds(jj * 16, 16)
                    iv = bidx_v[k, sl]
                    bg[0][k, sl] = c * N_NODES + iv
                    bg[1][k, sl] = c * (N_REL * NPAD) + iv
                    bg[2][k, sl] = c * (N_REL * NPAD) + NPAD + iv
                    bg[3][k, sl] = iv
                    bg[4][k, sl] = NPAD + iv
            gat = []
            for k in range(2):
                dsl = pl.ds(k * CHUNK, CHUNK)
                gat += [
                    _Op(d2_hbm.at[bg[0].at[k]], dD.at[dsl], sem_t),
                    _Op(s_out.at[bg[1].at[k]], s0r.at[dsl], sem_t),
                    _Op(s_out.at[bg[2].at[k]], s1r.at[dsl], sem_t),
                    _Op(cnt_hbm.at[bg[3].at[k]], c0r.at[dsl], sem_t),
                    _Op(cnt_hbm.at[bg[4].at[k]], c1r.at[dsl], sem_t),
                ]
            for op in gat:
                op.start()
            for op in gat:
                op.wait()

            def comb(i, _):
                inv0 = 1.0 / jnp.maximum(c0r[i, pl.ds(0, 16)], 1.0)
                inv1 = 1.0 / jnp.maximum(c1r[i, pl.ds(0, 16)], 1.0)
                for k in range(d_half // 16):
                    sl = pl.ds(k * 16, 16)
                    br[i, sl] = (dD[i, sl] + s0r[i, sl] * inv0
                                 + s1r[i, sl] * inv1)
                return 0

            lax.fori_loop(0, _BPT, comb, 0)
            pltpu.sync_copy(br, p_out.at[pl.ds(c * N_BATCH + bb, _BPT)])

    return pl.kernel(body, out_type=out_type, mesh=_sc_mesh(),
                     scratch_types=scratch,
                     compiler_params=pltpu.CompilerParams(
                         use_tc_tiling_on_sc=False))


def _gather_kernel_body(table_hbm, idx_hbm, out_hbm, idx_v, rows_v, sem):
    wid = lax.axis_index("s") * NC + lax.axis_index("c")
    per_w = N_BATCH // (NC * NS)
    base = wid * per_w
    pltpu.sync_copy(idx_hbm.at[pl.ds(base, per_w)], idx_v)
    pltpu.async_copy(table_hbm.at[idx_v], rows_v, sem).wait()
    pltpu.sync_copy(rows_v, out_hbm.at[pl.ds(base, per_w)])


def _make_gather_kernel():
    per_w = N_BATCH // (NC * NS)
    return pl.kernel(
        _gather_kernel_body,
        out_type=jax.ShapeDtypeStruct((N_BATCH, D_EXP), jnp.float32),
        mesh=_sc_mesh(),
        scratch_types=[
            pltpu.VMEM((per_w,), jnp.int32),
            pltpu.VMEM((per_w, D_EXP), jnp.float32),
            pltpu.SemaphoreType.DMA,
        ],
        compiler_params=pltpu.CompilerParams(use_tc_tiling_on_sc=False),
    )


_BN = 1000
_H1 = D_HID // 2
_H2 = D_EXP // 2


def _transform1_body(x_ref, basis_ref, comp_ref, root_ref, bias_ref,
                     y_ref, dense_ref):
    x = x_ref[...]
    w0 = comp_ref[0, 0] * basis_ref[0] + comp_ref[0, 1] * basis_ref[1]
    w1 = comp_ref[1, 0] * basis_ref[0] + comp_ref[1, 1] * basis_ref[1]
    y0 = jnp.dot(x, w0, preferred_element_type=jnp.float32)
    y1 = jnp.dot(x, w1, preferred_element_type=jnp.float32)
    y_ref[0, 0] = y0[:, :_H1]
    y_ref[0, 1] = y1[:, :_H1]
    y_ref[1, 0] = y0[:, _H1:]
    y_ref[1, 1] = y1[:, _H1:]
    dense_ref[...] = (
        jnp.dot(x, root_ref[...], preferred_element_type=jnp.float32)
        + bias_ref[...]
    )


def _transform1(x, basis1, comp1, root1, bias1):
    grid = N_NODES // _BN
    return pl.pallas_call(
        _transform1_body,
        grid=(grid,),
        in_specs=[
            pl.BlockSpec((_BN, D_IN), lambda i: (i, 0)),
            pl.BlockSpec((N_REL, D_IN, D_HID), lambda i: (0, 0, 0)),
            pl.BlockSpec(memory_space=pltpu.SMEM),
            pl.BlockSpec((D_IN, D_HID), lambda i: (0, 0)),
            pl.BlockSpec((1, D_HID), lambda i: (0, 0)),
        ],
        out_specs=[
            pl.BlockSpec((NC, N_REL, _BN, _H1), lambda i: (0, 0, i, 0)),
            pl.BlockSpec((_BN, D_HID), lambda i: (i, 0)),
        ],
        out_shape=[
            jax.ShapeDtypeStruct((NC, N_REL, N_NODES, _H1), jnp.float32),
            jax.ShapeDtypeStruct((N_NODES, D_HID), jnp.float32),
        ],
    )(x, basis1, comp1, root1, bias1)


def _combine2_body(dense_ref, s00_ref, s01_ref, s10_ref, s11_ref,
                   c0_ref, c1_ref, basis_ref, comp_ref, root_ref, bias_ref,
                   y_ref, dense2_ref):
    i0 = 1.0 / jnp.maximum(c0_ref[:, :1], 1.0)
    i1 = 1.0 / jnp.maximum(c1_ref[:, :1], 1.0)
    h = dense_ref[...] + jnp.concatenate(
        [s00_ref[:, :_H1] * i0 + s01_ref[:, :_H1] * i1,
         s10_ref[:, :_H1] * i0 + s11_ref[:, :_H1] * i1], axis=1)
    h = jnp.maximum(h, 0.0)
    w0 = comp_ref[0, 0] * basis_ref[0] + comp_ref[0, 1] * basis_ref[1]
    w1 = comp_ref[1, 0] * basis_ref[0] + comp_ref[1, 1] * basis_ref[1]
    y0 = jnp.dot(h, w0, preferred_element_type=jnp.float32)
    y1 = jnp.dot(h, w1, preferred_element_type=jnp.float32)
    y_ref[0, 0] = y0[:, :_H2]
    y_ref[0, 1] = y1[:, :_H2]
    y_ref[1, 0] = y0[:, _H2:]
    y_ref[1, 1] = y1[:, _H2:]
    d2 = (jnp.dot(h, root_ref[...], preferred_element_type=jnp.float32)
          + bias_ref[...])
    dense2_ref[0] = d2[:, :_H2]
    dense2_ref[1] = d2[:, _H2:]


def _combine2(dense1, s00, s01, s10, s11, c0, c1, basis2, comp2, root2,
              bias2):
    grid = N_NODES // _BN
    half = pl.BlockSpec((_BN, 128), lambda i: (i, 0))
    cnt = pl.BlockSpec((_BN, 128), lambda i: (i, 0))
    return pl.pallas_call(
        _combine2_body,
        grid=(grid,),
        in_specs=[
            pl.BlockSpec((_BN, D_HID), lambda i: (i, 0)),
            half, half, half, half, cnt, cnt,
            pl.BlockSpec((N_REL, D_HID, D_EXP), lambda i: (0, 0, 0)),
            pl.BlockSpec(memory_space=pltpu.SMEM),
            pl.BlockSpec((D_HID, D_EXP), lambda i: (0, 0)),
            pl.BlockSpec((1, D_EXP), lambda i: (0, 0)),
        ],
        out_specs=[
            pl.BlockSpec((NC, N_REL, _BN, _H2), lambda i: (0, 0, i, 0)),
            pl.BlockSpec((NC, _BN, _H2), lambda i: (0, i, 0)),
        ],
        out_shape=[
            jax.ShapeDtypeStruct((NC, N_REL, N_NODES, _H2), jnp.float32),
            jax.ShapeDtypeStruct((NC, N_NODES, _H2), jnp.float32),
        ],
    )(dense1, s00, s01, s10, s11, c0, c1, basis2, comp2, root2, bias2)


def _classifier_body(p0_ref, p1_ref, w1_ref, b1_ref, w2_ref, b2_ref,
                     br_ref, out_ref):
    x = jnp.concatenate([p0_ref[...], p1_ref[...]], axis=1)
    br_ref[...] = x
    h = jnp.dot(x, w1_ref[...], preferred_element_type=jnp.float32)
    h = jnp.maximum(h + b1_ref[...], 0.0)
    z = jnp.dot(h, w2_ref[...], preferred_element_type=jnp.float32)
    out_ref[...] = jax.nn.sigmoid(z + b2_ref[...])


def _classifier(p0, p1, Wc1, bc1, Wc2, bc2):
    return pl.pallas_call(
        _classifier_body,
        in_specs=[
            pl.BlockSpec((N_BATCH, _H2), lambda: (0, 0)),
            pl.BlockSpec((N_BATCH, _H2), lambda: (0, 0)),
            pl.BlockSpec((D_EXP, 32), lambda: (0, 0)),
            pl.BlockSpec((1, 32), lambda: (0, 0)),
            pl.BlockSpec((32, 1), lambda: (0, 0)),
            pl.BlockSpec((1, 1), lambda: (0, 0)),
        ],
        out_specs=[
            pl.BlockSpec((N_BATCH, D_EXP), lambda: (0, 0)),
            pl.BlockSpec((N_BATCH, 1), lambda: (0, 0)),
        ],
        out_shape=[
            jax.ShapeDtypeStruct((N_BATCH, D_EXP), jnp.float32),
            jax.ShapeDtypeStruct((N_BATCH, 1), jnp.float32),
        ],
    )(p0, p1, Wc1, bc1, Wc2, bc2)


def kernel(node_indices, edge_index, edge_type, node_features, basis1, comp1,
           root1, bias1, basis2, comp2, root2, bias2, Wc1, bc1, Wc2, bc2):
    e3 = jnp.concatenate([edge_index, edge_type[None, :]], axis=0)

    zrows = N_REL * ROWS_PER_TILE // 4
    zb1 = jnp.zeros((zrows, _H1), jnp.float32)
    zb2 = jnp.zeros((zrows, _H2), jnp.float32)
    z16 = jnp.zeros((ROWS_PER_TILE, 16), jnp.float32)
    ones = jnp.ones((CHUNK, 16), jnp.float32)

    y1, dense1 = _transform1(node_features, basis1, comp1, root1,
                             bias1.reshape(1, D_HID))
    y1_flat = y1.reshape(NC * N_REL * N_NODES, _H1)
    s1_flat, cnt_w, cnt_flat = _make_edge_kernel(True, _H1, NBUF1)(
        e3, y1_flat, zb1, z16, ones)
    s1 = [s1_flat[k * NPAD:k * NPAD + N_NODES] for k in range(4)]
    c0 = cnt_w[:N_NODES]
    c1 = cnt_w[NPAD:NPAD + N_NODES]

    y2, dense2c = _combine2(dense1, s1[0], s1[1], s1[2], s1[3], c0, c1,
                            basis2, comp2, root2, bias2.reshape(1, D_EXP))
    y2_flat = y2.reshape(NC * N_REL * N_NODES, _H2)
    d2_flat = dense2c.reshape(NC * N_NODES, _H2)
    _, p_flat = _make_edge_kernel(False, _H2, NBUF2, final=True)(
        e3, y2_flat, zb2, node_indices, d2_flat, cnt_flat)

    batch_repr, bot_prob = _classifier(
        p_flat[:N_BATCH], p_flat[N_BATCH:], Wc1, bc1.reshape(1, 32), Wc2,
        bc2.reshape(1, 1))
    return (batch_repr, bot_prob)
